# Initial kernel scaffold; baseline (speedup 1.0000x reference)
import jax
import jax.numpy as jnp
from jax.experimental import pallas as pl


def kernel(xyz1, xyz2, points1, points2, W0, b0, gamma0, beta0, W1, b1, gamma1, beta1):
    raise NotImplementedError("write your pallas kernel here")


# speedup vs baseline: 21.3051x; 21.3051x over previous
import functools

import jax
import jax.numpy as jnp
import numpy as np
from jax import lax
from jax.experimental import pallas as pl
from jax.experimental.pallas import tpu as pltpu
from jax.experimental.pallas import tpu_sc as plsc

_B, _N, _S = 4, 8192, 2048
_D1, _D2 = 64, 128
_C0, _C1 = 256, 128
_TN = 256
_NT = _N // _TN
_BN = _B * _N
_CNT = float(_BN)
_ROWS = 3 * _BN
_NW = 32
_RPW = _ROWS // _NW
_CH = 512
_NCH = _RPW // _CH


def _knn_body(x1_ref, x2_ref, w_ref, idx_ref):
    b = pl.program_id(0)
    x1 = x1_ref[0]
    x2 = x2_ref[0]
    x1sq = jnp.sum(x1 * x1, axis=1, keepdims=True)
    x2sq = jnp.sum(x2 * x2, axis=0, keepdims=True)
    cross = lax.dot_general(x1, x2, (((1,), (0,)), ((), ())),
                            preferred_element_type=jnp.float32)
    d = x1sq - 2.0 * cross + x2sq
    iota = lax.broadcasted_iota(jnp.int32, (_TN, _S), 1)
    big = jnp.float32(np.inf)
    dks, iks = [], []
    dw = d
    for _ in range(3):
        mn = jnp.min(dw, axis=1, keepdims=True)
        hit = dw == mn
        ix = jnp.min(jnp.where(hit, iota, _S), axis=1, keepdims=True)
        dks.append(mn)
        iks.append(ix)
        dw = jnp.where(iota == ix, big, dw)
    d3 = jnp.concatenate(dks, axis=1)
    i3 = jnp.concatenate(iks, axis=1)
    recip = 1.0 / (d3 + 1e-8)
    w_ref[0] = recip / jnp.sum(recip, axis=1, keepdims=True)
    idx_ref[0] = i3 + b * _S


def _mlp1_body(p1_ref, g_ref, w_ref, w0a_ref, w0b_ref, b0_ref, h1_ref, st_ref):
    w = w_ref[0]
    interp = (g_ref[0] * w[:, 0:1]
              + g_ref[1] * w[:, 1:2]
              + g_ref[2] * w[:, 2:3])
    h = lax.dot_general(w0a_ref[...], p1_ref[0], (((1,), (0,)), ((), ())),
                        preferred_element_type=jnp.float32)
    h = h + lax.dot_general(w0b_ref[...], interp, (((1,), (1,)), ((), ())),
                            preferred_element_type=jnp.float32)
    h = h + b0_ref[...]
    h1_ref[0] = h

    @pl.when((pl.program_id(0) == 0) & (pl.program_id(1) == 0))
    def _():
        st_ref[...] = jnp.zeros_like(st_ref)

    st_ref[:, 0:1] += jnp.sum(h, axis=1, keepdims=True)
    st_ref[:, 1:2] += jnp.sum(h * h, axis=1, keepdims=True)


def _mlp2_body(h1_ref, st1_ref, w1_ref, b1_ref, g0_ref, be0_ref, h2_ref, st_ref):
    mean = st1_ref[:, 0:1] / _CNT
    var = st1_ref[:, 1:2] / _CNT - mean * mean
    scale = g0_ref[...] * lax.rsqrt(var + 1e-5)
    y = jnp.maximum((h1_ref[0] - mean) * scale + be0_ref[...], 0.0)
    h = lax.dot_general(w1_ref[...], y, (((1,), (0,)), ((), ())),
                        preferred_element_type=jnp.float32) + b1_ref[...]
    h2_ref[0] = h

    @pl.when((pl.program_id(0) == 0) & (pl.program_id(1) == 0))
    def _():
        st_ref[...] = jnp.zeros_like(st_ref)

    st_ref[:, 0:1] += jnp.sum(h, axis=1, keepdims=True)
    st_ref[:, 1:2] += jnp.sum(h * h, axis=1, keepdims=True)


def _bn2_body(h2_ref, st2_ref, g1_ref, be1_ref, out_ref):
    mean = st2_ref[:, 0:1] / _CNT
    var = st2_ref[:, 1:2] / _CNT - mean * mean
    scale = g1_ref[...] * lax.rsqrt(var + 1e-5)
    out_ref[0] = jnp.maximum((h2_ref[0] - mean) * scale + be1_ref[...], 0.0)


def _sc_gather(fidx, table):
    mesh = plsc.VectorSubcoreMesh(core_axis_name="c", subcore_axis_name="s")

    @functools.partial(
        pl.kernel,
        mesh=mesh,
        out_type=jax.ShapeDtypeStruct((_ROWS, _D2), jnp.float32),
        scratch_types=[
            pltpu.VMEM((_CH,), jnp.int32),
            pltpu.VMEM((_CH, _D2), jnp.float32),
            pltpu.SemaphoreType.DMA,
        ],
    )
    def gather_k(fidx_hbm, table_hbm, out_hbm, idx_v, rows_v, sem):
        wid = lax.axis_index("s") * 2 + lax.axis_index("c")
        base = wid * _RPW
        for j in range(_NCH):
            off = base + j * _CH
            pltpu.sync_copy(fidx_hbm.at[pl.ds(off, _CH)], idx_v)
            pltpu.async_copy(table_hbm.at[idx_v], rows_v, sem).wait()
            pltpu.sync_copy(rows_v, out_hbm.at[pl.ds(off, _CH)])

    return gather_k(fidx, table)


def kernel(xyz1, xyz2, points1, points2, W0, b0, gamma0, beta0, W1, b1,
           gamma1, beta1):
    x1t = jnp.transpose(xyz1, (0, 2, 1))

    w3, gidx = pl.pallas_call(
        _knn_body,
        grid=(_B, _NT),
        in_specs=[
            pl.BlockSpec((1, _TN, 3), lambda b, j: (b, j, 0)),
            pl.BlockSpec((1, 3, _S), lambda b, j: (b, 0, 0)),
        ],
        out_specs=[
            pl.BlockSpec((1, _TN, 3), lambda b, j: (b, j, 0)),
            pl.BlockSpec((1, _TN, 3), lambda b, j: (b, j, 0)),
        ],
        out_shape=[
            jax.ShapeDtypeStruct((_B, _N, 3), jnp.float32),
            jax.ShapeDtypeStruct((_B, _N, 3), jnp.int32),
        ],
        compiler_params=pltpu.CompilerParams(
            dimension_semantics=("arbitrary", "arbitrary")),
    )(x1t, xyz2)

    fidx = jnp.transpose(gidx, (2, 0, 1)).reshape(_ROWS)
    p2t = jnp.transpose(points2, (0, 2, 1)).reshape(_B * _S, _D2)
    g = _sc_gather(fidx, p2t).reshape(3, _BN, _D2)

    h1, st1 = pl.pallas_call(
        _mlp1_body,
        grid=(_B, _NT),
        in_specs=[
            pl.BlockSpec((1, _D1, _TN), lambda b, j: (b, 0, j)),
            pl.BlockSpec((3, _TN, _D2), lambda b, j: (0, b * _NT + j, 0)),
            pl.BlockSpec((1, _TN, 3), lambda b, j: (b, j, 0)),
            pl.BlockSpec((_C0, _D1), lambda b, j: (0, 0)),
            pl.BlockSpec((_C0, _D2), lambda b, j: (0, 0)),
            pl.BlockSpec((_C0, 1), lambda b, j: (0, 0)),
        ],
        out_specs=[
            pl.BlockSpec((1, _C0, _TN), lambda b, j: (b, 0, j)),
            pl.BlockSpec((_C0, 2), lambda b, j: (0, 0)),
        ],
        out_shape=[
            jax.ShapeDtypeStruct((_B, _C0, _N), jnp.float32),
            jax.ShapeDtypeStruct((_C0, 2), jnp.float32),
        ],
        compiler_params=pltpu.CompilerParams(
            dimension_semantics=("arbitrary", "arbitrary")),
    )(points1, g, w3, W0[:, :_D1], W0[:, _D1:], b0[:, None])

    h2, st2 = pl.pallas_call(
        _mlp2_body,
        grid=(_B, _NT),
        in_specs=[
            pl.BlockSpec((1, _C0, _TN), lambda b, j: (b, 0, j)),
            pl.BlockSpec((_C0, 2), lambda b, j: (0, 0)),
            pl.BlockSpec((_C1, _C0), lambda b, j: (0, 0)),
            pl.BlockSpec((_C1, 1), lambda b, j: (0, 0)),
            pl.BlockSpec((_C0, 1), lambda b, j: (0, 0)),
            pl.BlockSpec((_C0, 1), lambda b, j: (0, 0)),
        ],
        out_specs=[
            pl.BlockSpec((1, _C1, _TN), lambda b, j: (b, 0, j)),
            pl.BlockSpec((_C1, 2), lambda b, j: (0, 0)),
        ],
        out_shape=[
            jax.ShapeDtypeStruct((_B, _C1, _N), jnp.float32),
            jax.ShapeDtypeStruct((_C1, 2), jnp.float32),
        ],
        compiler_params=pltpu.CompilerParams(
            dimension_semantics=("arbitrary", "arbitrary")),
    )(h1, st1, W1, b1[:, None], gamma0[:, None], beta0[:, None])

    out = pl.pallas_call(
        _bn2_body,
        grid=(_B, _NT),
        in_specs=[
            pl.BlockSpec((1, _C1, _TN), lambda b, j: (b, 0, j)),
            pl.BlockSpec((_C1, 2), lambda b, j: (0, 0)),
            pl.BlockSpec((_C1, 1), lambda b, j: (0, 0)),
            pl.BlockSpec((_C1, 1), lambda b, j: (0, 0)),
        ],
        out_specs=pl.BlockSpec((1, _C1, _TN), lambda b, j: (b, 0, j)),
        out_shape=jax.ShapeDtypeStruct((_B, _C1, _N), jnp.float32),
        compiler_params=pltpu.CompilerParams(
            dimension_semantics=("arbitrary", "arbitrary")),
    )(h2, st2, gamma1[:, None], beta1[:, None])

    return out


# speedup vs baseline: 24.4242x; 1.1464x over previous
import functools

import jax
import jax.numpy as jnp
import numpy as np
from jax import lax
from jax.experimental import pallas as pl
from jax.experimental.pallas import tpu as pltpu
from jax.experimental.pallas import tpu_sc as plsc

_B, _N, _S = 4, 8192, 2048
_D1, _D2 = 64, 128
_C0, _C1 = 256, 128
_TN = 256
_NT = _N // _TN
_TNK = 512
_NTK = _N // _TNK
_BN = _B * _N
_CNT = float(_BN)
_ROWS = 3 * _BN
_NW = 32
_RPW = _ROWS // _NW
_CH = 512
_NCH = _RPW // _CH


def _knn_body(x1_ref, x2_ref, w_ref, idx_ref):
    b = pl.program_id(0)
    x1 = x1_ref[0]
    x2 = x2_ref[0]
    x1sq = jnp.sum(x1 * x1, axis=1, keepdims=True)
    x2sq = ((x2[0:1] * x2[0:1] + x2[1:2] * x2[1:2])
            + x2[2:3] * x2[2:3])
    cross = lax.dot_general(x1.astype(jnp.bfloat16), x2.astype(jnp.bfloat16),
                            (((1,), (0,)), ((), ())),
                            preferred_element_type=jnp.float32)
    d = (x1sq - 2.0 * cross) + x2sq
    dbits = lax.bitcast_convert_type(d, jnp.int32)
    intmax = jnp.int32(0x7FFFFFFF)
    m1 = jnp.full((_TNK, 128), intmax, jnp.int32)
    m2 = m1
    m3 = m1
    for c in range(_S // 128):
        x = lax.bitwise_or(
            lax.bitwise_and(dbits[:, c * 128:(c + 1) * 128],
                            jnp.int32(-16)), jnp.int32(c))
        hi1 = jnp.maximum(m1, x)
        m1 = jnp.minimum(m1, x)
        hi2 = jnp.maximum(m2, hi1)
        m2 = jnp.minimum(m2, hi1)
        m3 = jnp.minimum(m3, hi2)
    cand = jnp.concatenate([m1, m2, m3], axis=1)
    workf = lax.bitcast_convert_type(cand, jnp.float32)
    iotaf = lax.broadcasted_iota(jnp.int32, (_TNK, 384), 1).astype(jnp.float32)
    bigf = jnp.float32(3.0e38)
    dks, iks = [], []
    for _ in range(3):
        mn = jnp.min(workf, axis=1, keepdims=True)
        posf = jnp.min(jnp.where(workf == mn, iotaf, bigf),
                       axis=1, keepdims=True)
        workf = jnp.where(iotaf == posf, bigf, workf)
        mni = lax.bitcast_convert_type(mn, jnp.int32)
        pos = posf.astype(jnp.int32)
        dks.append(lax.bitwise_and(mni, jnp.int32(-16)))
        iks.append(lax.bitwise_and(mni, jnp.int32(15)) * 128
                   + lax.bitwise_and(pos, jnp.int32(127)))
    i3 = jnp.concatenate(iks, axis=1)
    d3 = lax.bitcast_convert_type(jnp.concatenate(dks, axis=1),
                                  jnp.float32)
    recip = 1.0 / (d3 + 1e-8)
    w_ref[0] = recip / jnp.sum(recip, axis=1, keepdims=True)
    idx_ref[0] = i3 + b * _S


def _mlp1_body(p1_ref, g_ref, w_ref, w0a_ref, w0b_ref, b0_ref, h1_ref, st_ref):
    w = w_ref[0]
    interp = (g_ref[0] * w[:, 0:1]
              + g_ref[1] * w[:, 1:2]
              + g_ref[2] * w[:, 2:3])
    h = lax.dot_general(w0a_ref[...], p1_ref[0], (((1,), (0,)), ((), ())),
                        preferred_element_type=jnp.float32)
    h = h + lax.dot_general(w0b_ref[...], interp, (((1,), (1,)), ((), ())),
                            preferred_element_type=jnp.float32)
    h = h + b0_ref[...]
    h1_ref[0] = h

    @pl.when((pl.program_id(0) == 0) & (pl.program_id(1) == 0))
    def _():
        st_ref[...] = jnp.zeros_like(st_ref)

    st_ref[:, 0:1] += jnp.sum(h, axis=1, keepdims=True)
    st_ref[:, 1:2] += jnp.sum(h * h, axis=1, keepdims=True)


def _mlp2_body(h1_ref, st1_ref, w1_ref, b1_ref, g0_ref, be0_ref, h2_ref, st_ref):
    mean = st1_ref[:, 0:1] / _CNT
    var = st1_ref[:, 1:2] / _CNT - mean * mean
    scale = g0_ref[...] * lax.rsqrt(var + 1e-5)
    y = jnp.maximum((h1_ref[0] - mean) * scale + be0_ref[...], 0.0)
    h = lax.dot_general(w1_ref[...], y, (((1,), (0,)), ((), ())),
                        preferred_element_type=jnp.float32) + b1_ref[...]
    h2_ref[0] = h

    @pl.when((pl.program_id(0) == 0) & (pl.program_id(1) == 0))
    def _():
        st_ref[...] = jnp.zeros_like(st_ref)

    st_ref[:, 0:1] += jnp.sum(h, axis=1, keepdims=True)
    st_ref[:, 1:2] += jnp.sum(h * h, axis=1, keepdims=True)


def _bn2_body(h2_ref, st2_ref, g1_ref, be1_ref, out_ref):
    mean = st2_ref[:, 0:1] / _CNT
    var = st2_ref[:, 1:2] / _CNT - mean * mean
    scale = g1_ref[...] * lax.rsqrt(var + 1e-5)
    out_ref[0] = jnp.maximum((h2_ref[0] - mean) * scale + be1_ref[...], 0.0)


def _sc_gather(fidx, table):
    mesh = plsc.VectorSubcoreMesh(core_axis_name="c", subcore_axis_name="s")

    @functools.partial(
        pl.kernel,
        mesh=mesh,
        out_type=jax.ShapeDtypeStruct((_ROWS, _D2), jnp.float32),
        scratch_types=[
            pltpu.VMEM((_CH,), jnp.int32),
            pltpu.VMEM((_CH, _D2), jnp.float32),
            pltpu.SemaphoreType.DMA,
        ],
    )
    def gather_k(fidx_hbm, table_hbm, out_hbm, idx_v, rows_v, sem):
        wid = lax.axis_index("s") * 2 + lax.axis_index("c")
        base = wid * _RPW
        for j in range(_NCH):
            off = base + j * _CH
            pltpu.sync_copy(fidx_hbm.at[pl.ds(off, _CH)], idx_v)
            pltpu.async_copy(table_hbm.at[idx_v], rows_v, sem).wait()
            pltpu.sync_copy(rows_v, out_hbm.at[pl.ds(off, _CH)])

    return gather_k(fidx, table)


def kernel(xyz1, xyz2, points1, points2, W0, b0, gamma0, beta0, W1, b1,
           gamma1, beta1):
    x1t = jnp.transpose(xyz1, (0, 2, 1))

    w3, gidx = pl.pallas_call(
        _knn_body,
        grid=(_B, _NTK),
        in_specs=[
            pl.BlockSpec((1, _TNK, 3), lambda b, j: (b, j, 0)),
            pl.BlockSpec((1, 3, _S), lambda b, j: (b, 0, 0)),
        ],
        out_specs=[
            pl.BlockSpec((1, _TNK, 3), lambda b, j: (b, j, 0)),
            pl.BlockSpec((1, _TNK, 3), lambda b, j: (b, j, 0)),
        ],
        out_shape=[
            jax.ShapeDtypeStruct((_B, _N, 3), jnp.float32),
            jax.ShapeDtypeStruct((_B, _N, 3), jnp.int32),
        ],
        compiler_params=pltpu.CompilerParams(
            dimension_semantics=("arbitrary", "arbitrary")),
    )(x1t, xyz2)

    fidx = jnp.transpose(gidx, (2, 0, 1)).reshape(_ROWS)
    p2t = jnp.transpose(points2, (0, 2, 1)).reshape(_B * _S, _D2)
    g = _sc_gather(fidx, p2t).reshape(3, _BN, _D2)

    h1, st1 = pl.pallas_call(
        _mlp1_body,
        grid=(_B, _NT),
        in_specs=[
            pl.BlockSpec((1, _D1, _TN), lambda b, j: (b, 0, j)),
            pl.BlockSpec((3, _TN, _D2), lambda b, j: (0, b * _NT + j, 0)),
            pl.BlockSpec((1, _TN, 3), lambda b, j: (b, j, 0)),
            pl.BlockSpec((_C0, _D1), lambda b, j: (0, 0)),
            pl.BlockSpec((_C0, _D2), lambda b, j: (0, 0)),
            pl.BlockSpec((_C0, 1), lambda b, j: (0, 0)),
        ],
        out_specs=[
            pl.BlockSpec((1, _C0, _TN), lambda b, j: (b, 0, j)),
            pl.BlockSpec((_C0, 2), lambda b, j: (0, 0)),
        ],
        out_shape=[
            jax.ShapeDtypeStruct((_B, _C0, _N), jnp.float32),
            jax.ShapeDtypeStruct((_C0, 2), jnp.float32),
        ],
        compiler_params=pltpu.CompilerParams(
            dimension_semantics=("arbitrary", "arbitrary")),
    )(points1, g, w3, W0[:, :_D1], W0[:, _D1:], b0[:, None])

    h2, st2 = pl.pallas_call(
        _mlp2_body,
        grid=(_B, _NT),
        in_specs=[
            pl.BlockSpec((1, _C0, _TN), lambda b, j: (b, 0, j)),
            pl.BlockSpec((_C0, 2), lambda b, j: (0, 0)),
            pl.BlockSpec((_C1, _C0), lambda b, j: (0, 0)),
            pl.BlockSpec((_C1, 1), lambda b, j: (0, 0)),
            pl.BlockSpec((_C0, 1), lambda b, j: (0, 0)),
            pl.BlockSpec((_C0, 1), lambda b, j: (0, 0)),
        ],
        out_specs=[
            pl.BlockSpec((1, _C1, _TN), lambda b, j: (b, 0, j)),
            pl.BlockSpec((_C1, 2), lambda b, j: (0, 0)),
        ],
        out_shape=[
            jax.ShapeDtypeStruct((_B, _C1, _N), jnp.float32),
            jax.ShapeDtypeStruct((_C1, 2), jnp.float32),
        ],
        compiler_params=pltpu.CompilerParams(
            dimension_semantics=("arbitrary", "arbitrary")),
    )(h1, st1, W1, b1[:, None], gamma0[:, None], beta0[:, None])

    out = pl.pallas_call(
        _bn2_body,
        grid=(_B, _NT),
        in_specs=[
            pl.BlockSpec((1, _C1, _TN), lambda b, j: (b, 0, j)),
            pl.BlockSpec((_C1, 2), lambda b, j: (0, 0)),
            pl.BlockSpec((_C1, 1), lambda b, j: (0, 0)),
            pl.BlockSpec((_C1, 1), lambda b, j: (0, 0)),
        ],
        out_specs=pl.BlockSpec((1, _C1, _TN), lambda b, j: (b, 0, j)),
        out_shape=jax.ShapeDtypeStruct((_B, _C1, _N), jnp.float32),
        compiler_params=pltpu.CompilerParams(
            dimension_semantics=("arbitrary", "arbitrary")),
    )(h2, st2, gamma1[:, None], beta1[:, None])

    return out


# speedup vs baseline: 31.2726x; 1.2804x over previous
import functools

import jax
import jax.numpy as jnp
import numpy as np
from jax import lax
from jax.experimental import pallas as pl
from jax.experimental.pallas import tpu as pltpu
from jax.experimental.pallas import tpu_sc as plsc

_B, _N, _S = 4, 8192, 2048
_D1, _D2 = 64, 128
_C0, _C1 = 256, 128
_TN = 512
_NT = _N // _TN
_TNK = 512
_NTK = _N // _TNK
_BN = _B * _N
_CNT = float(_BN)
_ROWS = 3 * _BN
_NW = 32
_RPW = _ROWS // _NW
_CH = 512
_NCH = _RPW // _CH


def _knn_body(x1_ref, x2_ref, w_ref, idx_ref):
    b = pl.program_id(0)
    x1 = x1_ref[0]
    x2 = x2_ref[0]
    x1sq = jnp.sum(x1 * x1, axis=1, keepdims=True)
    x2sq = ((x2[0:1] * x2[0:1] + x2[1:2] * x2[1:2])
            + x2[2:3] * x2[2:3])
    cross = lax.dot_general(x1.astype(jnp.bfloat16), x2.astype(jnp.bfloat16),
                            (((1,), (0,)), ((), ())),
                            preferred_element_type=jnp.float32)
    d = (x1sq - 2.0 * cross) + x2sq
    dbits = lax.bitcast_convert_type(d, jnp.int32)
    intmax = jnp.int32(0x7FFFFFFF)
    m1 = jnp.full((_TNK, 128), intmax, jnp.int32)
    m2 = m1
    m3 = m1
    for c in range(_S // 128):
        x = lax.bitwise_or(
            lax.bitwise_and(dbits[:, c * 128:(c + 1) * 128],
                            jnp.int32(-16)), jnp.int32(c))
        hi1 = jnp.maximum(m1, x)
        m1 = jnp.minimum(m1, x)
        hi2 = jnp.maximum(m2, hi1)
        m2 = jnp.minimum(m2, hi1)
        m3 = jnp.minimum(m3, hi2)
    cand = jnp.concatenate([m1, m2, m3], axis=1)
    workf = lax.bitcast_convert_type(cand, jnp.float32)
    iotaf = lax.broadcasted_iota(jnp.int32, (_TNK, 384), 1).astype(jnp.float32)
    bigf = jnp.float32(3.0e38)
    dks, iks = [], []
    for _ in range(3):
        mn = jnp.min(workf, axis=1, keepdims=True)
        posf = jnp.min(jnp.where(workf == mn, iotaf, bigf),
                       axis=1, keepdims=True)
        workf = jnp.where(iotaf == posf, bigf, workf)
        mni = lax.bitcast_convert_type(mn, jnp.int32)
        pos = posf.astype(jnp.int32)
        dks.append(lax.bitwise_and(mni, jnp.int32(-16)))
        iks.append(lax.bitwise_and(mni, jnp.int32(15)) * 128
                   + lax.bitwise_and(pos, jnp.int32(127)))
    i3 = jnp.concatenate(iks, axis=1)
    d3 = lax.bitcast_convert_type(jnp.concatenate(dks, axis=1),
                                  jnp.float32)
    recip = 1.0 / (d3 + 1e-8)
    w_ref[0] = recip / jnp.sum(recip, axis=1, keepdims=True)
    idx_ref[0] = i3 + b * _S


def _mlp1_body(p1_ref, g_ref, w_ref, w0a_ref, w0b_ref, b0_ref, h1_ref, st_ref):
    w = w_ref[0]
    interp = (g_ref[0] * w[:, 0:1]
              + g_ref[1] * w[:, 1:2]
              + g_ref[2] * w[:, 2:3])
    h = lax.dot_general(w0a_ref[...], p1_ref[0], (((1,), (0,)), ((), ())),
                        preferred_element_type=jnp.float32)
    h = h + lax.dot_general(w0b_ref[...], interp, (((1,), (1,)), ((), ())),
                            preferred_element_type=jnp.float32)
    h = h + b0_ref[...]
    h1_ref[0] = h

    @pl.when((pl.program_id(0) == 0) & (pl.program_id(1) == 0))
    def _():
        st_ref[...] = jnp.zeros_like(st_ref)

    st_ref[:, 0:1] += jnp.sum(h, axis=1, keepdims=True)
    st_ref[:, 1:2] += jnp.sum(h * h, axis=1, keepdims=True)


def _mlp2_body(h1_ref, st1_ref, w1_ref, b1_ref, g0_ref, be0_ref, h2_ref, st_ref):
    mean = st1_ref[:, 0:1] / _CNT
    var = st1_ref[:, 1:2] / _CNT - mean * mean
    scale = g0_ref[...] * lax.rsqrt(var + 1e-5)
    y = jnp.maximum((h1_ref[0] - mean) * scale + be0_ref[...], 0.0)
    h = lax.dot_general(w1_ref[...], y, (((1,), (0,)), ((), ())),
                        preferred_element_type=jnp.float32) + b1_ref[...]
    h2_ref[0] = h

    @pl.when((pl.program_id(0) == 0) & (pl.program_id(1) == 0))
    def _():
        st_ref[...] = jnp.zeros_like(st_ref)

    st_ref[:, 0:1] += jnp.sum(h, axis=1, keepdims=True)
    st_ref[:, 1:2] += jnp.sum(h * h, axis=1, keepdims=True)


def _bn2_body(h2_ref, st2_ref, g1_ref, be1_ref, out_ref):
    mean = st2_ref[:, 0:1] / _CNT
    var = st2_ref[:, 1:2] / _CNT - mean * mean
    scale = g1_ref[...] * lax.rsqrt(var + 1e-5)
    out_ref[0] = jnp.maximum((h2_ref[0] - mean) * scale + be1_ref[...], 0.0)


def _sc_gather(fidx, table):
    mesh = plsc.VectorSubcoreMesh(core_axis_name="c", subcore_axis_name="s")

    @functools.partial(
        pl.kernel,
        mesh=mesh,
        out_type=jax.ShapeDtypeStruct((_ROWS, _D2), jnp.float32),
        scratch_types=[
            pltpu.VMEM((_CH,), jnp.int32),
            pltpu.VMEM((_CH, _D2), jnp.float32),
            pltpu.SemaphoreType.DMA,
        ],
    )
    def gather_k(fidx_hbm, table_hbm, out_hbm, idx_v, rows_v, sem):
        wid = lax.axis_index("s") * 2 + lax.axis_index("c")
        base = wid * _RPW
        for j in range(_NCH):
            off = base + j * _CH
            pltpu.sync_copy(fidx_hbm.at[pl.ds(off, _CH)], idx_v)
            pltpu.async_copy(table_hbm.at[idx_v], rows_v, sem).wait()
            pltpu.sync_copy(rows_v, out_hbm.at[pl.ds(off, _CH)])

    return gather_k(fidx, table)


def kernel(xyz1, xyz2, points1, points2, W0, b0, gamma0, beta0, W1, b1,
           gamma1, beta1):
    x1t = jnp.transpose(xyz1, (0, 2, 1))

    w3, gidx = pl.pallas_call(
        _knn_body,
        grid=(_B, _NTK),
        in_specs=[
            pl.BlockSpec((1, _TNK, 3), lambda b, j: (b, j, 0)),
            pl.BlockSpec((1, 3, _S), lambda b, j: (b, 0, 0)),
        ],
        out_specs=[
            pl.BlockSpec((1, _TNK, 3), lambda b, j: (b, j, 0)),
            pl.BlockSpec((1, _TNK, 3), lambda b, j: (b, j, 0)),
        ],
        out_shape=[
            jax.ShapeDtypeStruct((_B, _N, 3), jnp.float32),
            jax.ShapeDtypeStruct((_B, _N, 3), jnp.int32),
        ],
        compiler_params=pltpu.CompilerParams(
            dimension_semantics=("arbitrary", "arbitrary")),
    )(x1t, xyz2)

    fidx = jnp.transpose(gidx, (2, 0, 1)).reshape(_ROWS)
    p2t = jnp.transpose(points2, (0, 2, 1)).reshape(_B * _S, _D2)
    g = _sc_gather(fidx, p2t).reshape(3, _BN, _D2)

    h1, st1 = pl.pallas_call(
        _mlp1_body,
        grid=(_B, _NT),
        in_specs=[
            pl.BlockSpec((1, _D1, _TN), lambda b, j: (b, 0, j)),
            pl.BlockSpec((3, _TN, _D2), lambda b, j: (0, b * _NT + j, 0)),
            pl.BlockSpec((1, _TN, 3), lambda b, j: (b, j, 0)),
            pl.BlockSpec((_C0, _D1), lambda b, j: (0, 0)),
            pl.BlockSpec((_C0, _D2), lambda b, j: (0, 0)),
            pl.BlockSpec((_C0, 1), lambda b, j: (0, 0)),
        ],
        out_specs=[
            pl.BlockSpec((1, _C0, _TN), lambda b, j: (b, 0, j)),
            pl.BlockSpec((_C0, 2), lambda b, j: (0, 0)),
        ],
        out_shape=[
            jax.ShapeDtypeStruct((_B, _C0, _N), jnp.float32),
            jax.ShapeDtypeStruct((_C0, 2), jnp.float32),
        ],
        compiler_params=pltpu.CompilerParams(
            dimension_semantics=("arbitrary", "arbitrary")),
    )(points1, g, w3, W0[:, :_D1], W0[:, _D1:], b0[:, None])

    h2, st2 = pl.pallas_call(
        _mlp2_body,
        grid=(_B, _NT),
        in_specs=[
            pl.BlockSpec((1, _C0, _TN), lambda b, j: (b, 0, j)),
            pl.BlockSpec((_C0, 2), lambda b, j: (0, 0)),
            pl.BlockSpec((_C1, _C0), lambda b, j: (0, 0)),
            pl.BlockSpec((_C1, 1), lambda b, j: (0, 0)),
            pl.BlockSpec((_C0, 1), lambda b, j: (0, 0)),
            pl.BlockSpec((_C0, 1), lambda b, j: (0, 0)),
        ],
        out_specs=[
            pl.BlockSpec((1, _C1, _TN), lambda b, j: (b, 0, j)),
            pl.BlockSpec((_C1, 2), lambda b, j: (0, 0)),
        ],
        out_shape=[
            jax.ShapeDtypeStruct((_B, _C1, _N), jnp.float32),
            jax.ShapeDtypeStruct((_C1, 2), jnp.float32),
        ],
        compiler_params=pltpu.CompilerParams(
            dimension_semantics=("arbitrary", "arbitrary")),
    )(h1, st1, W1, b1[:, None], gamma0[:, None], beta0[:, None])

    out = pl.pallas_call(
        _bn2_body,
        grid=(_B, _NT),
        in_specs=[
            pl.BlockSpec((1, _C1, _TN), lambda b, j: (b, 0, j)),
            pl.BlockSpec((_C1, 2), lambda b, j: (0, 0)),
            pl.BlockSpec((_C1, 1), lambda b, j: (0, 0)),
            pl.BlockSpec((_C1, 1), lambda b, j: (0, 0)),
        ],
        out_specs=pl.BlockSpec((1, _C1, _TN), lambda b, j: (b, 0, j)),
        out_shape=jax.ShapeDtypeStruct((_B, _C1, _N), jnp.float32),
        compiler_params=pltpu.CompilerParams(
            dimension_semantics=("arbitrary", "arbitrary")),
    )(h2, st2, gamma1[:, None], beta1[:, None])

    return out
